# baseline (speedup 1.0000x reference)
import functools

import jax
import jax.numpy as jnp
from jax import lax
from jax.experimental import pallas as pl
from jax.experimental.pallas import tpu as pltpu
from jax.experimental.pallas import tpu_sc as plsc

N = 10000
E = 160000
NH = 128
B = 1024

NC = 2
NS = 16
NW = NC * NS

EC = 320
CHUNKS = E // EC
SEG_ITERS = -(-CHUNKS // NW)
ROWS_PER_SUB = 624
TAIL_ROWS = N - NS * ROWS_PER_SUB
TAIL_BASE = NS * ROWS_PER_SUB

_sc_mesh = functools.partial(
    plsc.VectorSubcoreMesh,
    core_axis_name="c", subcore_axis_name="s",
    num_cores=NC, num_subcores=NS,
)


BM1 = 400


def _mm1_body(x_ref, w_ref, o_ref):
    o_ref[...] = jnp.maximum(
        jnp.dot(x_ref[...], w_ref[...], preferred_element_type=jnp.float32),
        0.0,
    )


def _h1(X, W1):
    return pl.pallas_call(
        _mm1_body,
        grid=(N // BM1,),
        in_specs=[
            pl.BlockSpec((BM1, N), lambda m: (m, 0)),
            pl.BlockSpec((N, NH), lambda m: (0, 0)),
        ],
        out_specs=pl.BlockSpec((BM1, NH), lambda m: (m, 0)),
        out_shape=jax.ShapeDtypeStruct((N, NH), jnp.float32),
    )(X, W1)


def _segsum_body(h_hbm, src_hbm, dst_hbm, zer_hbm, out0, out1,
                 srcv, dstv, rows, acc, sem):
    cid = lax.axis_index("c")
    sid = lax.axis_index("s")
    w = sid * NC + cid

    r0 = sid * ROWS_PER_SUB

    @pl.when(cid == 0)
    def _():
        pltpu.sync_copy(h_hbm.at[pl.ds(r0, ROWS_PER_SUB)],
                        acc.at[pl.ds(r0, ROWS_PER_SUB)])

        @pl.when(sid == NS - 1)
        def _():
            pltpu.sync_copy(h_hbm.at[pl.ds(TAIL_BASE, TAIL_ROWS)],
                            acc.at[pl.ds(TAIL_BASE, TAIL_ROWS)])

    @pl.when(cid != 0)
    def _():
        pltpu.sync_copy(zer_hbm.at[pl.ds(0, ROWS_PER_SUB)],
                        acc.at[pl.ds(r0, ROWS_PER_SUB)])

        @pl.when(sid == NS - 1)
        def _():
            pltpu.sync_copy(zer_hbm.at[pl.ds(0, TAIL_ROWS)],
                            acc.at[pl.ds(TAIL_BASE, TAIL_ROWS)])

    plsc.subcore_barrier()

    def body(i, _):
        chunk = i * NW + w

        @pl.when(chunk < CHUNKS)
        def _():
            base = chunk * EC
            pltpu.sync_copy(src_hbm.at[pl.ds(base, EC)], srcv)
            pltpu.sync_copy(dst_hbm.at[pl.ds(base, EC)], dstv)
            pltpu.async_copy(h_hbm.at[srcv], rows, sem).wait()
            pltpu.sync_copy(rows, acc.at[dstv], add=True)
        return 0

    lax.fori_loop(0, SEG_ITERS, body, 0)
    plsc.subcore_barrier()

    @pl.when(cid == 0)
    def _():
        pltpu.sync_copy(acc.at[pl.ds(r0, ROWS_PER_SUB)],
                        out0.at[pl.ds(r0, ROWS_PER_SUB)])

        @pl.when(sid == NS - 1)
        def _():
            pltpu.sync_copy(acc.at[pl.ds(TAIL_BASE, TAIL_ROWS)],
                            out0.at[pl.ds(TAIL_BASE, TAIL_ROWS)])

    @pl.when(cid != 0)
    def _():
        pltpu.sync_copy(acc.at[pl.ds(r0, ROWS_PER_SUB)],
                        out1.at[pl.ds(r0, ROWS_PER_SUB)])

        @pl.when(sid == NS - 1)
        def _():
            pltpu.sync_copy(acc.at[pl.ds(TAIL_BASE, TAIL_ROWS)],
                            out1.at[pl.ds(TAIL_BASE, TAIL_ROWS)])


def _segsum(h, src, dst, zer):
    k = pl.kernel(
        _segsum_body,
        out_type=(
            jax.ShapeDtypeStruct((N, NH), jnp.float32),
            jax.ShapeDtypeStruct((N, NH), jnp.float32),
        ),
        mesh=_sc_mesh(),
        scratch_types=[
            pltpu.VMEM((EC,), jnp.int32),
            pltpu.VMEM((EC,), jnp.int32),
            pltpu.VMEM((EC, NH), jnp.float32),
            pltpu.VMEM_SHARED((N, NH), jnp.float32),
            pltpu.SemaphoreType.DMA,
        ],
    )
    return k(h, src, dst, zer)


BM2 = 1000


def _mm2_body(a_ref, b_ref, w_ref, o_ref):
    agg = a_ref[...] + b_ref[...]
    o_ref[...] = jnp.maximum(
        jnp.dot(agg, w_ref[...], preferred_element_type=jnp.float32), 0.0)


def _h2(p0, p1, W2):
    return pl.pallas_call(
        _mm2_body,
        grid=(N // BM2,),
        in_specs=[
            pl.BlockSpec((BM2, NH), lambda m: (m, 0)),
            pl.BlockSpec((BM2, NH), lambda m: (m, 0)),
            pl.BlockSpec((NH, NH), lambda m: (0, 0)),
        ],
        out_specs=pl.BlockSpec((BM2, NH), lambda m: (m, 0)),
        out_shape=jax.ShapeDtypeStruct((N, NH), jnp.float32),
    )(p0, p1, W2)


DG = 32


def _drows_body(data_ref, d_ref, o_ref, sem):
    def outer(g, _):
        def start(k, _):
            b = g * DG + k
            pltpu.make_async_copy(
                d_ref.at[pl.ds(data_ref[b], 1)],
                o_ref.at[pl.ds(b, 1)],
                sem).start()
            return 0

        lax.fori_loop(0, DG, start, 0)

        def waitk(k, _):
            b = g * DG + k
            pltpu.make_async_copy(
                d_ref.at[pl.ds(data_ref[b], 1)],
                o_ref.at[pl.ds(b, 1)],
                sem).wait()
            return 0

        lax.fori_loop(0, DG, waitk, 0)
        return 0

    lax.fori_loop(0, B // DG, outer, 0)


def _drows(D, data):
    return pl.pallas_call(
        _drows_body,
        in_specs=[
            pl.BlockSpec(memory_space=pltpu.SMEM),
            pl.BlockSpec(memory_space=pltpu.MemorySpace.HBM),
        ],
        out_specs=pl.BlockSpec(memory_space=pltpu.MemorySpace.HBM),
        out_shape=jax.ShapeDtypeStruct((B, N), jnp.float32),
        scratch_shapes=[pltpu.SemaphoreType.DMA],
    )(data, D)


RPW = B // NW
HIST = 10240


def _rbgather_body(q0_hbm, q1_hbm, data_hbm, zer1_hbm, rb0, rb1, chist,
                   dv, r0, r1, dv128, onev, acc1, sem):
    cid = lax.axis_index("c")
    sid = lax.axis_index("s")
    w = sid * NC + cid
    base = w * RPW

    pltpu.sync_copy(data_hbm.at[pl.ds(base, RPW)], dv)
    pltpu.async_copy(q0_hbm.at[dv], r0, sem).wait()
    pltpu.async_copy(q1_hbm.at[dv], r1, sem).wait()
    pltpu.sync_copy(r0, rb0.at[pl.ds(base, RPW)])
    pltpu.sync_copy(r1, rb1.at[pl.ds(base, RPW)])

    @pl.when(jnp.logical_and(cid == 0, sid == 0))
    def _():
        pltpu.sync_copy(zer1_hbm, acc1)
        for j in range(8):
            onev[pl.ds(j * 16, 16)] = jnp.full((16,), 1.0, dtype=jnp.float32)
        for j in range(B // 128):
            pltpu.sync_copy(data_hbm.at[pl.ds(j * 128, 128)], dv128)
            pltpu.sync_copy(onev, acc1.at[dv128], add=True)
        pltpu.sync_copy(acc1, chist)


def _rbgather(q0, q1, data, zer1):
    k = pl.kernel(
        _rbgather_body,
        out_type=(
            jax.ShapeDtypeStruct((B, NH), jnp.float32),
            jax.ShapeDtypeStruct((B, NH), jnp.float32),
            jax.ShapeDtypeStruct((HIST,), jnp.float32),
        ),
        mesh=_sc_mesh(),
        scratch_types=[
            pltpu.VMEM((RPW,), jnp.int32),
            pltpu.VMEM((RPW, NH), jnp.float32),
            pltpu.VMEM((RPW, NH), jnp.float32),
            pltpu.VMEM((128,), jnp.int32),
            pltpu.VMEM((128,), jnp.float32),
            pltpu.VMEM_SHARED((HIST,), jnp.float32),
            pltpu.SemaphoreType.DMA,
        ],
    )
    return k(q0, q1, data, zer1)


BLK = 128


def _loss_body(a_ref, b_ref, q0_ref, q1_ref, dr_ref, c_ref, out_ref):
    i = pl.program_id(0)
    rb = a_ref[...] + b_ref[...]
    rep = q0_ref[...] + q1_ref[...]
    gram = lax.dot_general(
        rb, rep,
        dimension_numbers=(((1,), (1,)), ((), ())),
        preferred_element_type=jnp.float32,
    )
    m = jnp.abs(gram - dr_ref[...]) * c_ref[...]
    part = jnp.sum(m)

    @pl.when(i == 0)
    def _():
        out_ref[0, 0] = 0.0

    out_ref[0, 0] += part


def _loss(rb0, rb1, q0, q1, drows, c2d):
    return pl.pallas_call(
        _loss_body,
        grid=(B // BLK,),
        in_specs=[
            pl.BlockSpec((BLK, NH), lambda i: (i, 0)),
            pl.BlockSpec((BLK, NH), lambda i: (i, 0)),
            pl.BlockSpec((N, NH), lambda i: (0, 0)),
            pl.BlockSpec((N, NH), lambda i: (0, 0)),
            pl.BlockSpec((BLK, N), lambda i: (i, 0)),
            pl.BlockSpec((1, N), lambda i: (0, 0)),
        ],
        out_specs=pl.BlockSpec(memory_space=pltpu.SMEM),
        out_shape=jax.ShapeDtypeStruct((1, 1), jnp.float32),
    )(rb0, rb1, q0, q1, drows, c2d)


def kernel(data, X, D, edge_index, W1, W2):
    src = edge_index[0]
    dst = edge_index[1]
    zer = jnp.zeros((ROWS_PER_SUB, NH), jnp.float32)
    zer1 = jnp.zeros((HIST,), jnp.float32)

    dr = _drows(D, data)
    h1 = _h1(X, W1)
    p0, p1 = _segsum(h1, src, dst, zer)
    h2 = _h2(p0, p1, W2)
    q0, q1 = _segsum(h2, src, dst, zer)
    rb0, rb1, chist = _rbgather(q0, q1, data, zer1)
    c2d = chist[:N].reshape(1, N)
    return _loss(rb0, rb1, q0, q1, dr, c2d).reshape(1)


# speedup vs baseline: 3.8797x; 3.8797x over previous
import functools

import jax
import jax.numpy as jnp
from jax import lax
from jax.experimental import pallas as pl
from jax.experimental.pallas import tpu as pltpu
from jax.experimental.pallas import tpu_sc as plsc

N = 10000
E = 160000
NH = 128
B = 1024

NC = 2
NS = 16
NW = NC * NS

EC = 320
CHUNKS = E // EC
SEG_ITERS = -(-CHUNKS // NW)
ROWS_PER_SUB = 624
TAIL_ROWS = N - NS * ROWS_PER_SUB
TAIL_BASE = NS * ROWS_PER_SUB

_sc_mesh = functools.partial(
    plsc.VectorSubcoreMesh,
    core_axis_name="c", subcore_axis_name="s",
    num_cores=NC, num_subcores=NS,
)


BM1 = 400


def _mm1_body(x_ref, w_ref, o_ref):
    o_ref[...] = jnp.maximum(
        jnp.dot(x_ref[...], w_ref[...], preferred_element_type=jnp.float32),
        0.0,
    )


def _h1(X, W1):
    return pl.pallas_call(
        _mm1_body,
        grid=(N // BM1,),
        in_specs=[
            pl.BlockSpec((BM1, N), lambda m: (m, 0)),
            pl.BlockSpec((N, NH), lambda m: (0, 0)),
        ],
        out_specs=pl.BlockSpec((BM1, NH), lambda m: (m, 0)),
        out_shape=jax.ShapeDtypeStruct((N, NH), jnp.float32),
    )(X, W1)


def _segsum_body(h_hbm, src_hbm, dst_hbm, zer_hbm, out0, out1,
                 srcv, dstv, rows, acc, sem):
    cid = lax.axis_index("c")
    sid = lax.axis_index("s")
    w = sid * NC + cid

    r0 = sid * ROWS_PER_SUB

    @pl.when(cid == 0)
    def _():
        pltpu.sync_copy(h_hbm.at[pl.ds(r0, ROWS_PER_SUB)],
                        acc.at[pl.ds(r0, ROWS_PER_SUB)])

        @pl.when(sid == NS - 1)
        def _():
            pltpu.sync_copy(h_hbm.at[pl.ds(TAIL_BASE, TAIL_ROWS)],
                            acc.at[pl.ds(TAIL_BASE, TAIL_ROWS)])

    @pl.when(cid != 0)
    def _():
        pltpu.sync_copy(zer_hbm.at[pl.ds(0, ROWS_PER_SUB)],
                        acc.at[pl.ds(r0, ROWS_PER_SUB)])

        @pl.when(sid == NS - 1)
        def _():
            pltpu.sync_copy(zer_hbm.at[pl.ds(0, TAIL_ROWS)],
                            acc.at[pl.ds(TAIL_BASE, TAIL_ROWS)])

    plsc.subcore_barrier()

    def body(i, _):
        chunk = i * NW + w

        @pl.when(chunk < CHUNKS)
        def _():
            base = chunk * EC
            pltpu.sync_copy(src_hbm.at[pl.ds(base, EC)], srcv)
            pltpu.sync_copy(dst_hbm.at[pl.ds(base, EC)], dstv)
            pltpu.async_copy(h_hbm.at[srcv], rows, sem).wait()
            pltpu.sync_copy(rows, acc.at[dstv], add=True)
        return 0

    lax.fori_loop(0, SEG_ITERS, body, 0)
    plsc.subcore_barrier()

    @pl.when(cid == 0)
    def _():
        pltpu.sync_copy(acc.at[pl.ds(r0, ROWS_PER_SUB)],
                        out0.at[pl.ds(r0, ROWS_PER_SUB)])

        @pl.when(sid == NS - 1)
        def _():
            pltpu.sync_copy(acc.at[pl.ds(TAIL_BASE, TAIL_ROWS)],
                            out0.at[pl.ds(TAIL_BASE, TAIL_ROWS)])

    @pl.when(cid != 0)
    def _():
        pltpu.sync_copy(acc.at[pl.ds(r0, ROWS_PER_SUB)],
                        out1.at[pl.ds(r0, ROWS_PER_SUB)])

        @pl.when(sid == NS - 1)
        def _():
            pltpu.sync_copy(acc.at[pl.ds(TAIL_BASE, TAIL_ROWS)],
                            out1.at[pl.ds(TAIL_BASE, TAIL_ROWS)])


def _segsum(h, src, dst, zer):
    k = pl.kernel(
        _segsum_body,
        out_type=(
            jax.ShapeDtypeStruct((N, NH), jnp.float32),
            jax.ShapeDtypeStruct((N, NH), jnp.float32),
        ),
        mesh=_sc_mesh(),
        scratch_types=[
            pltpu.VMEM((EC,), jnp.int32),
            pltpu.VMEM((EC,), jnp.int32),
            pltpu.VMEM((EC, NH), jnp.float32),
            pltpu.VMEM_SHARED((N, NH), jnp.float32),
            pltpu.SemaphoreType.DMA,
        ],
    )
    return k(h, src, dst, zer)


BM2 = 1000


def _mm2_body(a_ref, b_ref, w_ref, o_ref):
    agg = a_ref[...] + b_ref[...]
    o_ref[...] = jnp.maximum(
        jnp.dot(agg, w_ref[...], preferred_element_type=jnp.float32), 0.0)


def _h2(p0, p1, W2):
    return pl.pallas_call(
        _mm2_body,
        grid=(N // BM2,),
        in_specs=[
            pl.BlockSpec((BM2, NH), lambda m: (m, 0)),
            pl.BlockSpec((BM2, NH), lambda m: (m, 0)),
            pl.BlockSpec((NH, NH), lambda m: (0, 0)),
        ],
        out_specs=pl.BlockSpec((BM2, NH), lambda m: (m, 0)),
        out_shape=jax.ShapeDtypeStruct((N, NH), jnp.float32),
    )(p0, p1, W2)


RPW = B // NW
HIST = 10240
NMAIN = 9984
NTAIL = N - NMAIN
CW = 3328
NWIN = NMAIN // CW
TPW = B * NTAIL // NW


def _dgather_body(d_hbm, tflat_hbm, data_hbm, fit_hbm, zer1_hbm,
                  drows, dtail, chist,
                  dv, buf, tiv, tvv, dv128, onev, acc1, sem):
    cid = lax.axis_index("c")
    sid = lax.axis_index("s")
    w = sid * NC + cid
    base = w * RPW

    pltpu.sync_copy(data_hbm.at[pl.ds(base, RPW)], dv)
    for wi in range(NWIN):
        pltpu.async_copy(d_hbm.at[dv, pl.ds(wi * CW, CW)], buf, sem).wait()
        pltpu.sync_copy(buf, drows.at[pl.ds(base, RPW), pl.ds(wi * CW, CW)])

    pltpu.sync_copy(fit_hbm.at[pl.ds(w * TPW, TPW)], tiv)
    pltpu.async_copy(tflat_hbm.at[tiv], tvv, sem).wait()
    pltpu.sync_copy(tvv, dtail.at[pl.ds(w * TPW, TPW)])

    @pl.when(jnp.logical_and(cid == 0, sid == 0))
    def _():
        pltpu.sync_copy(zer1_hbm, acc1)
        for j in range(8):
            onev[pl.ds(j * 16, 16)] = jnp.full((16,), 1.0, dtype=jnp.float32)
        for j in range(B // 128):
            pltpu.sync_copy(data_hbm.at[pl.ds(j * 128, 128)], dv128)
            pltpu.sync_copy(onev, acc1.at[dv128], add=True)
        pltpu.sync_copy(acc1, chist)


def _dgather(D, tflat, data, fit, zer1):
    k = pl.kernel(
        _dgather_body,
        out_type=(
            jax.ShapeDtypeStruct((B, NMAIN), jnp.float32),
            jax.ShapeDtypeStruct((B * NTAIL,), jnp.float32),
            jax.ShapeDtypeStruct((HIST,), jnp.float32),
        ),
        mesh=_sc_mesh(),
        scratch_types=[
            pltpu.VMEM((RPW,), jnp.int32),
            pltpu.VMEM((RPW, CW), jnp.float32),
            pltpu.VMEM((TPW,), jnp.int32),
            pltpu.VMEM((TPW,), jnp.float32),
            pltpu.VMEM((128,), jnp.int32),
            pltpu.VMEM((128,), jnp.float32),
            pltpu.VMEM_SHARED((HIST,), jnp.float32),
            pltpu.SemaphoreType.DMA,
        ],
    )
    return k(D, tflat, data, fit, zer1)


def _rbgather_body(q0_hbm, q1_hbm, data_hbm, rb0, rb1, dv, r0, r1, sem):
    cid = lax.axis_index("c")
    sid = lax.axis_index("s")
    w = sid * NC + cid
    base = w * RPW

    pltpu.sync_copy(data_hbm.at[pl.ds(base, RPW)], dv)
    pltpu.async_copy(q0_hbm.at[dv], r0, sem).wait()
    pltpu.async_copy(q1_hbm.at[dv], r1, sem).wait()
    pltpu.sync_copy(r0, rb0.at[pl.ds(base, RPW)])
    pltpu.sync_copy(r1, rb1.at[pl.ds(base, RPW)])


def _rbgather(q0, q1, data):
    k = pl.kernel(
        _rbgather_body,
        out_type=(
            jax.ShapeDtypeStruct((B, NH), jnp.float32),
            jax.ShapeDtypeStruct((B, NH), jnp.float32),
        ),
        mesh=_sc_mesh(),
        scratch_types=[
            pltpu.VMEM((RPW,), jnp.int32),
            pltpu.VMEM((RPW, NH), jnp.float32),
            pltpu.VMEM((RPW, NH), jnp.float32),
            pltpu.SemaphoreType.DMA,
        ],
    )
    return k(q0, q1, data)


BLK = 128


def _loss_body(a_ref, b_ref, q0_ref, q1_ref, dr_ref, dt_ref, c_ref, ct_ref,
               out_ref):
    i = pl.program_id(0)
    rb = a_ref[...] + b_ref[...]
    rep = q0_ref[...] + q1_ref[...]
    gram = lax.dot_general(
        rb, rep,
        dimension_numbers=(((1,), (1,)), ((), ())),
        preferred_element_type=jnp.float32,
    )
    m = jnp.abs(gram[:, :NMAIN] - dr_ref[...]) * c_ref[...]
    mt = jnp.abs(gram[:, NMAIN:] - dt_ref[...]) * ct_ref[...]
    part = jnp.sum(m) + jnp.sum(mt)

    @pl.when(i == 0)
    def _():
        out_ref[0, 0] = 0.0

    out_ref[0, 0] += part


def _loss(rb0, rb1, q0, q1, drows, dtail, c2d, ct2d):
    return pl.pallas_call(
        _loss_body,
        grid=(B // BLK,),
        in_specs=[
            pl.BlockSpec((BLK, NH), lambda i: (i, 0)),
            pl.BlockSpec((BLK, NH), lambda i: (i, 0)),
            pl.BlockSpec((N, NH), lambda i: (0, 0)),
            pl.BlockSpec((N, NH), lambda i: (0, 0)),
            pl.BlockSpec((BLK, NMAIN), lambda i: (i, 0)),
            pl.BlockSpec((BLK, NTAIL), lambda i: (i, 0)),
            pl.BlockSpec((1, NMAIN), lambda i: (0, 0)),
            pl.BlockSpec((1, NTAIL), lambda i: (0, 0)),
        ],
        out_specs=pl.BlockSpec(memory_space=pltpu.SMEM),
        out_shape=jax.ShapeDtypeStruct((1, 1), jnp.float32),
    )(rb0, rb1, q0, q1, drows, dtail, c2d, ct2d)


def kernel(data, X, D, edge_index, W1, W2):
    src = edge_index[0]
    dst = edge_index[1]
    zer = jnp.zeros((ROWS_PER_SUB, NH), jnp.float32)
    zer1 = jnp.zeros((HIST,), jnp.float32)

    tflat = D[:, NMAIN:].reshape(-1)
    fit = (data[:, None] * NTAIL + jnp.arange(NTAIL, dtype=jnp.int32)[None, :]
           ).reshape(-1)

    dr, dtail, chist = _dgather(D, tflat, data, fit, zer1)
    h1 = _h1(X, W1)
    p0, p1 = _segsum(h1, src, dst, zer)
    h2 = _h2(p0, p1, W2)
    q0, q1 = _segsum(h2, src, dst, zer)
    rb0, rb1 = _rbgather(q0, q1, data)
    c2d = chist[:NMAIN].reshape(1, NMAIN)
    ct2d = chist[NMAIN:N].reshape(1, NTAIL)
    return _loss(rb0, rb1, q0, q1, dr, dtail.reshape(B, NTAIL),
                 c2d, ct2d).reshape(1)


# speedup vs baseline: 4.3623x; 1.1244x over previous
import functools

import jax
import jax.numpy as jnp
from jax import lax
from jax.experimental import pallas as pl
from jax.experimental.pallas import tpu as pltpu
from jax.experimental.pallas import tpu_sc as plsc

N = 10000
E = 160000
NH = 128
B = 1024

NC = 2
NS = 16
NW = NC * NS

EPW = E // NW
EC = 192
SEG_ITERS = EPW // EC
ETAIL = EPW - SEG_ITERS * EC
ROWS_PER_SUB = 624
TAIL_ROWS = N - NS * ROWS_PER_SUB
TAIL_BASE = NS * ROWS_PER_SUB

_sc_mesh = functools.partial(
    plsc.VectorSubcoreMesh,
    core_axis_name="c", subcore_axis_name="s",
    num_cores=NC, num_subcores=NS,
)


BM1 = 400


def _mm1_body(x_ref, w_ref, o_ref):
    o_ref[...] = jnp.maximum(
        jnp.dot(x_ref[...], w_ref[...], preferred_element_type=jnp.float32),
        0.0,
    )


def _h1(X, W1):
    return pl.pallas_call(
        _mm1_body,
        grid=(N // BM1,),
        in_specs=[
            pl.BlockSpec((BM1, N), lambda m: (m, 0)),
            pl.BlockSpec((N, NH), lambda m: (0, 0)),
        ],
        out_specs=pl.BlockSpec((BM1, NH), lambda m: (m, 0)),
        out_shape=jax.ShapeDtypeStruct((N, NH), jnp.float32),
    )(X, W1)


def _segsum_body(h_hbm, src_hbm, dst_hbm, zer_hbm, out0, out1,
                 srcva, dstva, srcvb, dstvb, rowsa, rowsb, acc, sema, semb):
    cid = lax.axis_index("c")
    sid = lax.axis_index("s")
    w = sid * NC + cid

    r0 = sid * ROWS_PER_SUB

    @pl.when(cid == 0)
    def _():
        pltpu.sync_copy(h_hbm.at[pl.ds(r0, ROWS_PER_SUB)],
                        acc.at[pl.ds(r0, ROWS_PER_SUB)])

        @pl.when(sid == NS - 1)
        def _():
            pltpu.sync_copy(h_hbm.at[pl.ds(TAIL_BASE, TAIL_ROWS)],
                            acc.at[pl.ds(TAIL_BASE, TAIL_ROWS)])

    @pl.when(cid != 0)
    def _():
        pltpu.sync_copy(zer_hbm.at[pl.ds(0, ROWS_PER_SUB)],
                        acc.at[pl.ds(r0, ROWS_PER_SUB)])

        @pl.when(sid == NS - 1)
        def _():
            pltpu.sync_copy(zer_hbm.at[pl.ds(0, TAIL_ROWS)],
                            acc.at[pl.ds(TAIL_BASE, TAIL_ROWS)])

    plsc.subcore_barrier()

    ebase = w * EPW
    bufs = (((srcva, dstva), rowsa, sema), ((srcvb, dstvb), rowsb, semb))
    sizes = [EC] * SEG_ITERS + [ETAIL]

    def _launch(c):
        iv, rows, sem = bufs[c % 2]
        n = sizes[c]
        off = ebase + c * EC
        pltpu.sync_copy(src_hbm.at[pl.ds(off, n)], iv[0].at[pl.ds(0, n)])
        pltpu.sync_copy(dst_hbm.at[pl.ds(off, n)], iv[1].at[pl.ds(0, n)])
        return pltpu.async_copy(h_hbm.at[iv[0].at[pl.ds(0, n)]],
                                rows.at[pl.ds(0, n)], sem)

    cps = [None] * (SEG_ITERS + 1)
    cps[0] = _launch(0)
    for c in range(SEG_ITERS + 1):
        if c + 1 <= SEG_ITERS:
            cps[c + 1] = _launch(c + 1)
        iv, rows, _ = bufs[c % 2]
        n = sizes[c]
        cps[c].wait()
        pltpu.sync_copy(rows.at[pl.ds(0, n)],
                        acc.at[iv[1].at[pl.ds(0, n)]], add=True)

    plsc.subcore_barrier()

    @pl.when(cid == 0)
    def _():
        pltpu.sync_copy(acc.at[pl.ds(r0, ROWS_PER_SUB)],
                        out0.at[pl.ds(r0, ROWS_PER_SUB)])

        @pl.when(sid == NS - 1)
        def _():
            pltpu.sync_copy(acc.at[pl.ds(TAIL_BASE, TAIL_ROWS)],
                            out0.at[pl.ds(TAIL_BASE, TAIL_ROWS)])

    @pl.when(cid != 0)
    def _():
        pltpu.sync_copy(acc.at[pl.ds(r0, ROWS_PER_SUB)],
                        out1.at[pl.ds(r0, ROWS_PER_SUB)])

        @pl.when(sid == NS - 1)
        def _():
            pltpu.sync_copy(acc.at[pl.ds(TAIL_BASE, TAIL_ROWS)],
                            out1.at[pl.ds(TAIL_BASE, TAIL_ROWS)])


def _segsum(h, src, dst, zer):
    k = pl.kernel(
        _segsum_body,
        out_type=(
            jax.ShapeDtypeStruct((N, NH), jnp.float32),
            jax.ShapeDtypeStruct((N, NH), jnp.float32),
        ),
        mesh=_sc_mesh(),
        scratch_types=[
            pltpu.VMEM((EC,), jnp.int32),
            pltpu.VMEM((EC,), jnp.int32),
            pltpu.VMEM((EC,), jnp.int32),
            pltpu.VMEM((EC,), jnp.int32),
            pltpu.VMEM((EC, NH), jnp.float32),
            pltpu.VMEM((EC, NH), jnp.float32),
            pltpu.VMEM_SHARED((N, NH), jnp.float32),
            pltpu.SemaphoreType.DMA,
            pltpu.SemaphoreType.DMA,
        ],
    )
    return k(h, src, dst, zer)


BM2 = 1000


def _mm2_body(a_ref, b_ref, w_ref, o_ref):
    agg = a_ref[...] + b_ref[...]
    o_ref[...] = jnp.maximum(
        jnp.dot(agg, w_ref[...], preferred_element_type=jnp.float32), 0.0)


def _h2(p0, p1, W2):
    return pl.pallas_call(
        _mm2_body,
        grid=(N // BM2,),
        in_specs=[
            pl.BlockSpec((BM2, NH), lambda m: (m, 0)),
            pl.BlockSpec((BM2, NH), lambda m: (m, 0)),
            pl.BlockSpec((NH, NH), lambda m: (0, 0)),
        ],
        out_specs=pl.BlockSpec((BM2, NH), lambda m: (m, 0)),
        out_shape=jax.ShapeDtypeStruct((N, NH), jnp.float32),
    )(p0, p1, W2)


RPW = B // NW
HIST = 10240
NMAIN = 9984
NTAIL = N - NMAIN
CW = 3328
NWIN = NMAIN // CW
TPW = B * NTAIL // NW


def _dgather_body(d_hbm, tflat_hbm, data_hbm, fit_hbm, zer1_hbm,
                  drows, dtail, chist,
                  dv, buf, tiv, tvv, dv128, onev, acc1, sem):
    cid = lax.axis_index("c")
    sid = lax.axis_index("s")
    w = sid * NC + cid
    base = w * RPW

    pltpu.sync_copy(data_hbm.at[pl.ds(base, RPW)], dv)
    for wi in range(NWIN):
        pltpu.async_copy(d_hbm.at[dv, pl.ds(wi * CW, CW)], buf, sem).wait()
        pltpu.sync_copy(buf, drows.at[pl.ds(base, RPW), pl.ds(wi * CW, CW)])

    pltpu.sync_copy(fit_hbm.at[pl.ds(w * TPW, TPW)], tiv)
    pltpu.async_copy(tflat_hbm.at[tiv], tvv, sem).wait()
    pltpu.sync_copy(tvv, dtail.at[pl.ds(w * TPW, TPW)])

    @pl.when(jnp.logical_and(cid == 0, sid == 0))
    def _():
        pltpu.sync_copy(zer1_hbm, acc1)
        for j in range(8):
            onev[pl.ds(j * 16, 16)] = jnp.full((16,), 1.0, dtype=jnp.float32)
        for j in range(B // 128):
            pltpu.sync_copy(data_hbm.at[pl.ds(j * 128, 128)], dv128)
            pltpu.sync_copy(onev, acc1.at[dv128], add=True)
        pltpu.sync_copy(acc1, chist)


def _dgather(D, tflat, data, fit, zer1):
    k = pl.kernel(
        _dgather_body,
        out_type=(
            jax.ShapeDtypeStruct((B, NMAIN), jnp.float32),
            jax.ShapeDtypeStruct((B * NTAIL,), jnp.float32),
            jax.ShapeDtypeStruct((HIST,), jnp.float32),
        ),
        mesh=_sc_mesh(),
        scratch_types=[
            pltpu.VMEM((RPW,), jnp.int32),
            pltpu.VMEM((RPW, CW), jnp.float32),
            pltpu.VMEM((TPW,), jnp.int32),
            pltpu.VMEM((TPW,), jnp.float32),
            pltpu.VMEM((128,), jnp.int32),
            pltpu.VMEM((128,), jnp.float32),
            pltpu.VMEM_SHARED((HIST,), jnp.float32),
            pltpu.SemaphoreType.DMA,
        ],
    )
    return k(D, tflat, data, fit, zer1)


def _rbgather_body(q0_hbm, q1_hbm, data_hbm, rb0, rb1, dv, r0, r1, sem):
    cid = lax.axis_index("c")
    sid = lax.axis_index("s")
    w = sid * NC + cid
    base = w * RPW

    pltpu.sync_copy(data_hbm.at[pl.ds(base, RPW)], dv)
    pltpu.async_copy(q0_hbm.at[dv], r0, sem).wait()
    pltpu.async_copy(q1_hbm.at[dv], r1, sem).wait()
    pltpu.sync_copy(r0, rb0.at[pl.ds(base, RPW)])
    pltpu.sync_copy(r1, rb1.at[pl.ds(base, RPW)])


def _rbgather(q0, q1, data):
    k = pl.kernel(
        _rbgather_body,
        out_type=(
            jax.ShapeDtypeStruct((B, NH), jnp.float32),
            jax.ShapeDtypeStruct((B, NH), jnp.float32),
        ),
        mesh=_sc_mesh(),
        scratch_types=[
            pltpu.VMEM((RPW,), jnp.int32),
            pltpu.VMEM((RPW, NH), jnp.float32),
            pltpu.VMEM((RPW, NH), jnp.float32),
            pltpu.SemaphoreType.DMA,
        ],
    )
    return k(q0, q1, data)


BLK = 128


def _loss_body(a_ref, b_ref, q0_ref, q1_ref, dr_ref, dt_ref, c_ref, ct_ref,
               out_ref):
    i = pl.program_id(0)
    rb = a_ref[...] + b_ref[...]
    rep = q0_ref[...] + q1_ref[...]
    gram = lax.dot_general(
        rb, rep,
        dimension_numbers=(((1,), (1,)), ((), ())),
        preferred_element_type=jnp.float32,
    )
    m = jnp.abs(gram[:, :NMAIN] - dr_ref[...]) * c_ref[...]
    mt = jnp.abs(gram[:, NMAIN:] - dt_ref[...]) * ct_ref[...]
    part = jnp.sum(m) + jnp.sum(mt)

    @pl.when(i == 0)
    def _():
        out_ref[0, 0] = 0.0

    out_ref[0, 0] += part


def _loss(rb0, rb1, q0, q1, drows, dtail, c2d, ct2d):
    return pl.pallas_call(
        _loss_body,
        grid=(B // BLK,),
        in_specs=[
            pl.BlockSpec((BLK, NH), lambda i: (i, 0)),
            pl.BlockSpec((BLK, NH), lambda i: (i, 0)),
            pl.BlockSpec((N, NH), lambda i: (0, 0)),
            pl.BlockSpec((N, NH), lambda i: (0, 0)),
            pl.BlockSpec((BLK, NMAIN), lambda i: (i, 0)),
            pl.BlockSpec((BLK, NTAIL), lambda i: (i, 0)),
            pl.BlockSpec((1, NMAIN), lambda i: (0, 0)),
            pl.BlockSpec((1, NTAIL), lambda i: (0, 0)),
        ],
        out_specs=pl.BlockSpec(memory_space=pltpu.SMEM),
        out_shape=jax.ShapeDtypeStruct((1, 1), jnp.float32),
    )(rb0, rb1, q0, q1, drows, dtail, c2d, ct2d)


def kernel(data, X, D, edge_index, W1, W2):
    src = edge_index[0]
    dst = edge_index[1]
    zer = jnp.zeros((ROWS_PER_SUB, NH), jnp.float32)
    zer1 = jnp.zeros((HIST,), jnp.float32)

    tflat = D[:, NMAIN:].reshape(-1)
    fit = (data[:, None] * NTAIL + jnp.arange(NTAIL, dtype=jnp.int32)[None, :]
           ).reshape(-1)

    dr, dtail, chist = _dgather(D, tflat, data, fit, zer1)
    h1 = _h1(X, W1)
    p0, p1 = _segsum(h1, src, dst, zer)
    h2 = _h2(p0, p1, W2)
    q0, q1 = _segsum(h2, src, dst, zer)
    rb0, rb1 = _rbgather(q0, q1, data)
    c2d = chist[:NMAIN].reshape(1, NMAIN)
    ct2d = chist[NMAIN:N].reshape(1, NTAIL)
    return _loss(rb0, rb1, q0, q1, dr, dtail.reshape(B, NTAIL),
                 c2d, ct2d).reshape(1)


# speedup vs baseline: 4.3654x; 1.0007x over previous
import functools

import jax
import jax.numpy as jnp
from jax import lax
from jax.experimental import pallas as pl
from jax.experimental.pallas import tpu as pltpu
from jax.experimental.pallas import tpu_sc as plsc

N = 10000
E = 160000
NH = 128
B = 1024

NC = 2
NS = 16
NW = NC * NS

EPW = E // NW
EC = 192
SEG_ITERS = EPW // EC
ETAIL = EPW - SEG_ITERS * EC
ROWS_PER_SUB = 624
TAIL_ROWS = N - NS * ROWS_PER_SUB
TAIL_BASE = NS * ROWS_PER_SUB

_sc_mesh = functools.partial(
    plsc.VectorSubcoreMesh,
    core_axis_name="c", subcore_axis_name="s",
    num_cores=NC, num_subcores=NS,
)


BM1 = 400


def _mm1_body(x_ref, w_ref, o_ref):
    o_ref[...] = jnp.maximum(
        jnp.dot(x_ref[...].astype(jnp.bfloat16),
                w_ref[...].astype(jnp.bfloat16),
                preferred_element_type=jnp.float32),
        0.0,
    )


def _h1(X, W1):
    return pl.pallas_call(
        _mm1_body,
        grid=(N // BM1,),
        in_specs=[
            pl.BlockSpec((BM1, N), lambda m: (m, 0)),
            pl.BlockSpec((N, NH), lambda m: (0, 0)),
        ],
        out_specs=pl.BlockSpec((BM1, NH), lambda m: (m, 0)),
        out_shape=jax.ShapeDtypeStruct((N, NH), jnp.float32),
    )(X, W1)


def _segsum_body(h_hbm, src_hbm, dst_hbm, zer_hbm, out0, out1,
                 srcva, dstva, srcvb, dstvb, rowsa, rowsb, acc, sema, semb):
    cid = lax.axis_index("c")
    sid = lax.axis_index("s")
    w = sid * NC + cid

    r0 = sid * ROWS_PER_SUB

    @pl.when(cid == 0)
    def _():
        pltpu.sync_copy(h_hbm.at[pl.ds(r0, ROWS_PER_SUB)],
                        acc.at[pl.ds(r0, ROWS_PER_SUB)])

        @pl.when(sid == NS - 1)
        def _():
            pltpu.sync_copy(h_hbm.at[pl.ds(TAIL_BASE, TAIL_ROWS)],
                            acc.at[pl.ds(TAIL_BASE, TAIL_ROWS)])

    @pl.when(cid != 0)
    def _():
        pltpu.sync_copy(zer_hbm.at[pl.ds(0, ROWS_PER_SUB)],
                        acc.at[pl.ds(r0, ROWS_PER_SUB)])

        @pl.when(sid == NS - 1)
        def _():
            pltpu.sync_copy(zer_hbm.at[pl.ds(0, TAIL_ROWS)],
                            acc.at[pl.ds(TAIL_BASE, TAIL_ROWS)])

    plsc.subcore_barrier()

    ebase = w * EPW
    bufs = (((srcva, dstva), rowsa, sema), ((srcvb, dstvb), rowsb, semb))
    sizes = [EC] * SEG_ITERS + [ETAIL]

    def _launch(c):
        iv, rows, sem = bufs[c % 2]
        n = sizes[c]
        off = ebase + c * EC
        pltpu.sync_copy(src_hbm.at[pl.ds(off, n)], iv[0].at[pl.ds(0, n)])
        pltpu.sync_copy(dst_hbm.at[pl.ds(off, n)], iv[1].at[pl.ds(0, n)])
        return pltpu.async_copy(h_hbm.at[iv[0].at[pl.ds(0, n)]],
                                rows.at[pl.ds(0, n)], sem)

    cps = [None] * (SEG_ITERS + 1)
    cps[0] = _launch(0)
    for c in range(SEG_ITERS + 1):
        if c + 1 <= SEG_ITERS:
            cps[c + 1] = _launch(c + 1)
        iv, rows, _ = bufs[c % 2]
        n = sizes[c]
        cps[c].wait()
        pltpu.sync_copy(rows.at[pl.ds(0, n)],
                        acc.at[iv[1].at[pl.ds(0, n)]], add=True)

    plsc.subcore_barrier()

    @pl.when(cid == 0)
    def _():
        pltpu.sync_copy(acc.at[pl.ds(r0, ROWS_PER_SUB)],
                        out0.at[pl.ds(r0, ROWS_PER_SUB)])

        @pl.when(sid == NS - 1)
        def _():
            pltpu.sync_copy(acc.at[pl.ds(TAIL_BASE, TAIL_ROWS)],
                            out0.at[pl.ds(TAIL_BASE, TAIL_ROWS)])

    @pl.when(cid != 0)
    def _():
        pltpu.sync_copy(acc.at[pl.ds(r0, ROWS_PER_SUB)],
                        out1.at[pl.ds(r0, ROWS_PER_SUB)])

        @pl.when(sid == NS - 1)
        def _():
            pltpu.sync_copy(acc.at[pl.ds(TAIL_BASE, TAIL_ROWS)],
                            out1.at[pl.ds(TAIL_BASE, TAIL_ROWS)])


def _segsum(h, src, dst, zer):
    k = pl.kernel(
        _segsum_body,
        out_type=(
            jax.ShapeDtypeStruct((N, NH), jnp.float32),
            jax.ShapeDtypeStruct((N, NH), jnp.float32),
        ),
        mesh=_sc_mesh(),
        scratch_types=[
            pltpu.VMEM((EC,), jnp.int32),
            pltpu.VMEM((EC,), jnp.int32),
            pltpu.VMEM((EC,), jnp.int32),
            pltpu.VMEM((EC,), jnp.int32),
            pltpu.VMEM((EC, NH), jnp.float32),
            pltpu.VMEM((EC, NH), jnp.float32),
            pltpu.VMEM_SHARED((N, NH), jnp.float32),
            pltpu.SemaphoreType.DMA,
            pltpu.SemaphoreType.DMA,
        ],
    )
    return k(h, src, dst, zer)


BM2 = 1000


def _mm2_body(a_ref, b_ref, w_ref, o_ref):
    agg = a_ref[...] + b_ref[...]
    o_ref[...] = jnp.maximum(
        jnp.dot(agg, w_ref[...], preferred_element_type=jnp.float32), 0.0)


def _h2(p0, p1, W2):
    return pl.pallas_call(
        _mm2_body,
        grid=(N // BM2,),
        in_specs=[
            pl.BlockSpec((BM2, NH), lambda m: (m, 0)),
            pl.BlockSpec((BM2, NH), lambda m: (m, 0)),
            pl.BlockSpec((NH, NH), lambda m: (0, 0)),
        ],
        out_specs=pl.BlockSpec((BM2, NH), lambda m: (m, 0)),
        out_shape=jax.ShapeDtypeStruct((N, NH), jnp.float32),
    )(p0, p1, W2)


RPW = B // NW
HIST = 10240
NMAIN = 9984
NTAIL = N - NMAIN
CW = 3328
NWIN = NMAIN // CW
TPW = B * NTAIL // NW


def _dgather_body(d_hbm, tflat_hbm, data_hbm, fit_hbm, zer1_hbm,
                  drows, dtail, chist,
                  dv, buf, tiv, tvv, dv128, onev, acc1, sem):
    cid = lax.axis_index("c")
    sid = lax.axis_index("s")
    w = sid * NC + cid
    base = w * RPW

    pltpu.sync_copy(data_hbm.at[pl.ds(base, RPW)], dv)
    for wi in range(NWIN):
        pltpu.async_copy(d_hbm.at[dv, pl.ds(wi * CW, CW)], buf, sem).wait()
        pltpu.sync_copy(buf, drows.at[pl.ds(base, RPW), pl.ds(wi * CW, CW)])

    pltpu.sync_copy(fit_hbm.at[pl.ds(w * TPW, TPW)], tiv)
    pltpu.async_copy(tflat_hbm.at[tiv], tvv, sem).wait()
    pltpu.sync_copy(tvv, dtail.at[pl.ds(w * TPW, TPW)])

    @pl.when(jnp.logical_and(cid == 0, sid == 0))
    def _():
        pltpu.sync_copy(zer1_hbm, acc1)
        for j in range(8):
            onev[pl.ds(j * 16, 16)] = jnp.full((16,), 1.0, dtype=jnp.float32)
        for j in range(B // 128):
            pltpu.sync_copy(data_hbm.at[pl.ds(j * 128, 128)], dv128)
            pltpu.sync_copy(onev, acc1.at[dv128], add=True)
        pltpu.sync_copy(acc1, chist)


def _dgather(D, tflat, data, fit, zer1):
    k = pl.kernel(
        _dgather_body,
        out_type=(
            jax.ShapeDtypeStruct((B, NMAIN), jnp.float32),
            jax.ShapeDtypeStruct((B * NTAIL,), jnp.float32),
            jax.ShapeDtypeStruct((HIST,), jnp.float32),
        ),
        mesh=_sc_mesh(),
        scratch_types=[
            pltpu.VMEM((RPW,), jnp.int32),
            pltpu.VMEM((RPW, CW), jnp.float32),
            pltpu.VMEM((TPW,), jnp.int32),
            pltpu.VMEM((TPW,), jnp.float32),
            pltpu.VMEM((128,), jnp.int32),
            pltpu.VMEM((128,), jnp.float32),
            pltpu.VMEM_SHARED((HIST,), jnp.float32),
            pltpu.SemaphoreType.DMA,
        ],
    )
    return k(D, tflat, data, fit, zer1)


def _rbgather_body(q0_hbm, q1_hbm, data_hbm, rb0, rb1, dv, r0, r1, sem):
    cid = lax.axis_index("c")
    sid = lax.axis_index("s")
    w = sid * NC + cid
    base = w * RPW

    pltpu.sync_copy(data_hbm.at[pl.ds(base, RPW)], dv)
    pltpu.async_copy(q0_hbm.at[dv], r0, sem).wait()
    pltpu.async_copy(q1_hbm.at[dv], r1, sem).wait()
    pltpu.sync_copy(r0, rb0.at[pl.ds(base, RPW)])
    pltpu.sync_copy(r1, rb1.at[pl.ds(base, RPW)])


def _rbgather(q0, q1, data):
    k = pl.kernel(
        _rbgather_body,
        out_type=(
            jax.ShapeDtypeStruct((B, NH), jnp.float32),
            jax.ShapeDtypeStruct((B, NH), jnp.float32),
        ),
        mesh=_sc_mesh(),
        scratch_types=[
            pltpu.VMEM((RPW,), jnp.int32),
            pltpu.VMEM((RPW, NH), jnp.float32),
            pltpu.VMEM((RPW, NH), jnp.float32),
            pltpu.SemaphoreType.DMA,
        ],
    )
    return k(q0, q1, data)


BLK = 128


def _loss_body(a_ref, b_ref, q0_ref, q1_ref, dr_ref, dt_ref, c_ref, ct_ref,
               out_ref):
    i = pl.program_id(0)
    rb = a_ref[...] + b_ref[...]
    rep = q0_ref[...] + q1_ref[...]
    gram = lax.dot_general(
        rb, rep,
        dimension_numbers=(((1,), (1,)), ((), ())),
        preferred_element_type=jnp.float32,
    )
    m = jnp.abs(gram[:, :NMAIN] - dr_ref[...]) * c_ref[...]
    mt = jnp.abs(gram[:, NMAIN:] - dt_ref[...]) * ct_ref[...]
    part = jnp.sum(m) + jnp.sum(mt)

    @pl.when(i == 0)
    def _():
        out_ref[0, 0] = 0.0

    out_ref[0, 0] += part


def _loss(rb0, rb1, q0, q1, drows, dtail, c2d, ct2d):
    return pl.pallas_call(
        _loss_body,
        grid=(B // BLK,),
        in_specs=[
            pl.BlockSpec((BLK, NH), lambda i: (i, 0)),
            pl.BlockSpec((BLK, NH), lambda i: (i, 0)),
            pl.BlockSpec((N, NH), lambda i: (0, 0)),
            pl.BlockSpec((N, NH), lambda i: (0, 0)),
            pl.BlockSpec((BLK, NMAIN), lambda i: (i, 0)),
            pl.BlockSpec((BLK, NTAIL), lambda i: (i, 0)),
            pl.BlockSpec((1, NMAIN), lambda i: (0, 0)),
            pl.BlockSpec((1, NTAIL), lambda i: (0, 0)),
        ],
        out_specs=pl.BlockSpec(memory_space=pltpu.SMEM),
        out_shape=jax.ShapeDtypeStruct((1, 1), jnp.float32),
    )(rb0, rb1, q0, q1, drows, dtail, c2d, ct2d)


def kernel(data, X, D, edge_index, W1, W2):
    src = edge_index[0]
    dst = edge_index[1]
    zer = jnp.zeros((ROWS_PER_SUB, NH), jnp.float32)
    zer1 = jnp.zeros((HIST,), jnp.float32)

    tflat = D[:, NMAIN:].reshape(-1)
    fit = (data[:, None] * NTAIL + jnp.arange(NTAIL, dtype=jnp.int32)[None, :]
           ).reshape(-1)

    dr, dtail, chist = _dgather(D, tflat, data, fit, zer1)
    h1 = _h1(X, W1)
    p0, p1 = _segsum(h1, src, dst, zer)
    h2 = _h2(p0, p1, W2)
    q0, q1 = _segsum(h2, src, dst, zer)
    rb0, rb1 = _rbgather(q0, q1, data)
    c2d = chist[:NMAIN].reshape(1, NMAIN)
    ct2d = chist[NMAIN:N].reshape(1, NTAIL)
    return _loss(rb0, rb1, q0, q1, dr, dtail.reshape(B, NTAIL),
                 c2d, ct2d).reshape(1)


# speedup vs baseline: 4.3716x; 1.0014x over previous
import functools

import jax
import jax.numpy as jnp
from jax import lax
from jax.experimental import pallas as pl
from jax.experimental.pallas import tpu as pltpu
from jax.experimental.pallas import tpu_sc as plsc

N = 10000
E = 160000
NH = 128
B = 1024

NC = 2
NS = 16
NW = NC * NS

EPW = E // NW
EC = 192
SEG_ITERS = EPW // EC
ETAIL = EPW - SEG_ITERS * EC
ROWS_PER_SUB = 624
TAIL_ROWS = N - NS * ROWS_PER_SUB
TAIL_BASE = NS * ROWS_PER_SUB

_sc_mesh = functools.partial(
    plsc.VectorSubcoreMesh,
    core_axis_name="c", subcore_axis_name="s",
    num_cores=NC, num_subcores=NS,
)


BM1 = 400


def _mm1_body(x_ref, w_ref, o_ref):
    o_ref[...] = jnp.maximum(
        jnp.dot(x_ref[...], w_ref[...], preferred_element_type=jnp.float32),
        0.0,
    )


def _h1(X, W1):
    return pl.pallas_call(
        _mm1_body,
        grid=(N // BM1,),
        in_specs=[
            pl.BlockSpec((BM1, N), lambda m: (m, 0)),
            pl.BlockSpec((N, NH), lambda m: (0, 0)),
        ],
        out_specs=pl.BlockSpec((BM1, NH), lambda m: (m, 0)),
        out_shape=jax.ShapeDtypeStruct((N, NH), jnp.float32),
    )(X, W1)


def _segsum_body(h_hbm, src_hbm, dst_hbm, zer_hbm, out0, out1,
                 srcva, dstva, srcvb, dstvb, rowsa, rowsb, acc, sema, semb):
    cid = lax.axis_index("c")
    sid = lax.axis_index("s")
    w = sid * NC + cid

    r0 = sid * ROWS_PER_SUB

    @pl.when(cid == 0)
    def _():
        pltpu.sync_copy(h_hbm.at[pl.ds(r0, ROWS_PER_SUB)],
                        acc.at[pl.ds(r0, ROWS_PER_SUB)])

        @pl.when(sid == NS - 1)
        def _():
            pltpu.sync_copy(h_hbm.at[pl.ds(TAIL_BASE, TAIL_ROWS)],
                            acc.at[pl.ds(TAIL_BASE, TAIL_ROWS)])

    @pl.when(cid != 0)
    def _():
        pltpu.sync_copy(zer_hbm.at[pl.ds(0, ROWS_PER_SUB)],
                        acc.at[pl.ds(r0, ROWS_PER_SUB)])

        @pl.when(sid == NS - 1)
        def _():
            pltpu.sync_copy(zer_hbm.at[pl.ds(0, TAIL_ROWS)],
                            acc.at[pl.ds(TAIL_BASE, TAIL_ROWS)])

    plsc.subcore_barrier()

    ebase = w * EPW
    bufs = (((srcva, dstva), rowsa, sema), ((srcvb, dstvb), rowsb, semb))
    sizes = [EC] * SEG_ITERS + [ETAIL]

    def _launch(c):
        iv, rows, sem = bufs[c % 2]
        n = sizes[c]
        off = ebase + c * EC
        pltpu.sync_copy(src_hbm.at[pl.ds(off, n)], iv[0].at[pl.ds(0, n)])
        pltpu.sync_copy(dst_hbm.at[pl.ds(off, n)], iv[1].at[pl.ds(0, n)])
        return pltpu.async_copy(h_hbm.at[iv[0].at[pl.ds(0, n)]],
                                rows.at[pl.ds(0, n)], sem)

    cps = [None] * (SEG_ITERS + 1)
    cps[0] = _launch(0)
    for c in range(SEG_ITERS + 1):
        if c + 1 <= SEG_ITERS:
            cps[c + 1] = _launch(c + 1)
        iv, rows, _ = bufs[c % 2]
        n = sizes[c]
        cps[c].wait()
        pltpu.sync_copy(rows.at[pl.ds(0, n)],
                        acc.at[iv[1].at[pl.ds(0, n)]], add=True)

    plsc.subcore_barrier()

    @pl.when(cid == 0)
    def _():
        pltpu.sync_copy(acc.at[pl.ds(r0, ROWS_PER_SUB)],
                        out0.at[pl.ds(r0, ROWS_PER_SUB)])

        @pl.when(sid == NS - 1)
        def _():
            pltpu.sync_copy(acc.at[pl.ds(TAIL_BASE, TAIL_ROWS)],
                            out0.at[pl.ds(TAIL_BASE, TAIL_ROWS)])

    @pl.when(cid != 0)
    def _():
        pltpu.sync_copy(acc.at[pl.ds(r0, ROWS_PER_SUB)],
                        out1.at[pl.ds(r0, ROWS_PER_SUB)])

        @pl.when(sid == NS - 1)
        def _():
            pltpu.sync_copy(acc.at[pl.ds(TAIL_BASE, TAIL_ROWS)],
                            out1.at[pl.ds(TAIL_BASE, TAIL_ROWS)])


def _segsum(h, src, dst, zer):
    k = pl.kernel(
        _segsum_body,
        out_type=(
            jax.ShapeDtypeStruct((N, NH), jnp.float32),
            jax.ShapeDtypeStruct((N, NH), jnp.float32),
        ),
        mesh=_sc_mesh(),
        scratch_types=[
            pltpu.VMEM((EC,), jnp.int32),
            pltpu.VMEM((EC,), jnp.int32),
            pltpu.VMEM((EC,), jnp.int32),
            pltpu.VMEM((EC,), jnp.int32),
            pltpu.VMEM((EC, NH), jnp.float32),
            pltpu.VMEM((EC, NH), jnp.float32),
            pltpu.VMEM_SHARED((N, NH), jnp.float32),
            pltpu.SemaphoreType.DMA,
            pltpu.SemaphoreType.DMA,
        ],
    )
    return k(h, src, dst, zer)


BM2 = 1000


def _mm2_body(a_ref, b_ref, w_ref, o_ref):
    agg = a_ref[...] + b_ref[...]
    o_ref[...] = jnp.maximum(
        jnp.dot(agg, w_ref[...], preferred_element_type=jnp.float32), 0.0)


def _h2(p0, p1, W2):
    return pl.pallas_call(
        _mm2_body,
        grid=(N // BM2,),
        in_specs=[
            pl.BlockSpec((BM2, NH), lambda m: (m, 0)),
            pl.BlockSpec((BM2, NH), lambda m: (m, 0)),
            pl.BlockSpec((NH, NH), lambda m: (0, 0)),
        ],
        out_specs=pl.BlockSpec((BM2, NH), lambda m: (m, 0)),
        out_shape=jax.ShapeDtypeStruct((N, NH), jnp.float32),
    )(p0, p1, W2)


RPW = B // NW
HIST = 10240
NMAIN = 9984
NTAIL = N - NMAIN
CW = 3328
NWIN = NMAIN // CW
TPW = B * NTAIL // NW


def _dgather_body(d_hbm, tflat_hbm, data_hbm, fit_hbm, zer1_hbm,
                  drows, dtail, chist,
                  dv, buf, tiv, tvv, dv128, onev, acc1, sem):
    cid = lax.axis_index("c")
    sid = lax.axis_index("s")
    w = sid * NC + cid
    base = w * RPW

    pltpu.sync_copy(data_hbm.at[pl.ds(base, RPW)], dv)
    for wi in range(NWIN):
        pltpu.async_copy(d_hbm.at[dv, pl.ds(wi * CW, CW)], buf, sem).wait()
        pltpu.sync_copy(buf, drows.at[pl.ds(base, RPW), pl.ds(wi * CW, CW)])

    pltpu.sync_copy(fit_hbm.at[pl.ds(w * TPW, TPW)], tiv)
    pltpu.async_copy(tflat_hbm.at[tiv], tvv, sem).wait()
    pltpu.sync_copy(tvv, dtail.at[pl.ds(w * TPW, TPW)])

    @pl.when(jnp.logical_and(cid == 0, sid == 0))
    def _():
        pltpu.sync_copy(zer1_hbm, acc1)
        for j in range(8):
            onev[pl.ds(j * 16, 16)] = jnp.full((16,), 1.0, dtype=jnp.float32)
        for j in range(B // 128):
            pltpu.sync_copy(data_hbm.at[pl.ds(j * 128, 128)], dv128)
            pltpu.sync_copy(onev, acc1.at[dv128], add=True)
        pltpu.sync_copy(acc1, chist)


def _dgather(D, tflat, data, fit, zer1):
    k = pl.kernel(
        _dgather_body,
        out_type=(
            jax.ShapeDtypeStruct((B, NMAIN), jnp.float32),
            jax.ShapeDtypeStruct((B * NTAIL,), jnp.float32),
            jax.ShapeDtypeStruct((HIST,), jnp.float32),
        ),
        mesh=_sc_mesh(),
        scratch_types=[
            pltpu.VMEM((RPW,), jnp.int32),
            pltpu.VMEM((RPW, CW), jnp.float32),
            pltpu.VMEM((TPW,), jnp.int32),
            pltpu.VMEM((TPW,), jnp.float32),
            pltpu.VMEM((128,), jnp.int32),
            pltpu.VMEM((128,), jnp.float32),
            pltpu.VMEM_SHARED((HIST,), jnp.float32),
            pltpu.SemaphoreType.DMA,
        ],
    )
    return k(D, tflat, data, fit, zer1)


def _rbgather_body(q0_hbm, q1_hbm, data_hbm, rb0, rb1, dv, r0, r1, sem):
    cid = lax.axis_index("c")
    sid = lax.axis_index("s")
    w = sid * NC + cid
    base = w * RPW

    pltpu.sync_copy(data_hbm.at[pl.ds(base, RPW)], dv)
    pltpu.async_copy(q0_hbm.at[dv], r0, sem).wait()
    pltpu.async_copy(q1_hbm.at[dv], r1, sem).wait()
    pltpu.sync_copy(r0, rb0.at[pl.ds(base, RPW)])
    pltpu.sync_copy(r1, rb1.at[pl.ds(base, RPW)])


def _rbgather(q0, q1, data):
    k = pl.kernel(
        _rbgather_body,
        out_type=(
            jax.ShapeDtypeStruct((B, NH), jnp.float32),
            jax.ShapeDtypeStruct((B, NH), jnp.float32),
        ),
        mesh=_sc_mesh(),
        scratch_types=[
            pltpu.VMEM((RPW,), jnp.int32),
            pltpu.VMEM((RPW, NH), jnp.float32),
            pltpu.VMEM((RPW, NH), jnp.float32),
            pltpu.SemaphoreType.DMA,
        ],
    )
    return k(q0, q1, data)


BLK = 128


def _loss_body(a_ref, b_ref, q0_ref, q1_ref, dr_ref, dt_ref, c_ref, ct_ref,
               out_ref):
    i = pl.program_id(0)
    rb = a_ref[...] + b_ref[...]
    rep = q0_ref[...] + q1_ref[...]
    gram = lax.dot_general(
        rb, rep,
        dimension_numbers=(((1,), (1,)), ((), ())),
        preferred_element_type=jnp.float32,
    )
    m = jnp.abs(gram[:, :NMAIN] - dr_ref[...]) * c_ref[...]
    mt = jnp.abs(gram[:, NMAIN:] - dt_ref[...]) * ct_ref[...]
    part = jnp.sum(m) + jnp.sum(mt)

    @pl.when(i == 0)
    def _():
        out_ref[0, 0] = 0.0

    out_ref[0, 0] += part


def _loss(rb0, rb1, q0, q1, drows, dtail, c2d, ct2d):
    return pl.pallas_call(
        _loss_body,
        grid=(B // BLK,),
        in_specs=[
            pl.BlockSpec((BLK, NH), lambda i: (i, 0)),
            pl.BlockSpec((BLK, NH), lambda i: (i, 0)),
            pl.BlockSpec((N, NH), lambda i: (0, 0)),
            pl.BlockSpec((N, NH), lambda i: (0, 0)),
            pl.BlockSpec((BLK, NMAIN), lambda i: (i, 0)),
            pl.BlockSpec((BLK, NTAIL), lambda i: (i, 0)),
            pl.BlockSpec((1, NMAIN), lambda i: (0, 0)),
            pl.BlockSpec((1, NTAIL), lambda i: (0, 0)),
        ],
        out_specs=pl.BlockSpec(memory_space=pltpu.SMEM),
        out_shape=jax.ShapeDtypeStruct((1, 1), jnp.float32),
    )(rb0, rb1, q0, q1, drows, dtail, c2d, ct2d)


def kernel(data, X, D, edge_index, W1, W2):
    src = edge_index[0]
    dst = edge_index[1]
    zer = jnp.zeros((ROWS_PER_SUB, NH), jnp.float32)
    zer1 = jnp.zeros((HIST,), jnp.float32)

    tflat = D[:, NMAIN:].reshape(-1)
    fit = (data[:, None] * NTAIL + jnp.arange(NTAIL, dtype=jnp.int32)[None, :]
           ).reshape(-1)

    h1 = _h1(X, W1)
    dr, dtail, chist = _dgather(D, tflat, data, fit, zer1)
    p0, p1 = _segsum(h1, src, dst, zer)
    h2 = _h2(p0, p1, W2)
    q0, q1 = _segsum(h2, src, dst, zer)
    rb0, rb1 = _rbgather(q0, q1, data)
    c2d = chist[:NMAIN].reshape(1, NMAIN)
    ct2d = chist[NMAIN:N].reshape(1, NTAIL)
    return _loss(rb0, rb1, q0, q1, dr, dtail.reshape(B, NTAIL),
                 c2d, ct2d).reshape(1)
